# Initial kernel scaffold; baseline (speedup 1.0000x reference)
import jax
import jax.numpy as jnp
from jax.experimental import pallas as pl


def kernel(x, position, edge_index, pseudo, batch, W1, root1, b1, W2, root2, b2, W3, root3, b3, fc_w, fc_b):
    raise NotImplementedError("write your pallas kernel here")


# speedup vs baseline: 4.0842x; 4.0842x over previous
import functools

import jax
import jax.numpy as jnp
from jax import lax
from jax.experimental import pallas as pl
from jax.experimental.pallas import tpu as pltpu
from jax.experimental.pallas import tpu_sc as plsc

K = 5
KK = K * K
N = 10000
E = 160000
NG = 64
N_PAD = 10240
E_PAD = 163840
NW = 32
EW = E_PAD // NW
B = 64
NB = EW // B
ROWS = E_PAD // 128
VROWS = E // 128
WROWS = ROWS // NW
NSL = N_PAD // 16


def _prep(p0, p1, src):
    def body(p0_r, p1_r, src_r, w00_r, w01_r, w10_r, w11_r, val_r,
             i0_r, i1_r, i2_r, i3_r):
        i = pl.program_id(0)
        rows = i * 128 + lax.broadcasted_iota(jnp.int32, (128, 128), 0)
        val = (rows < VROWS).astype(jnp.float32)
        a = p0_r[...] * (K - 1.0)
        ia = jnp.clip(jnp.floor(a), 0.0, K - 2.0)
        fa = a - ia
        b = p1_r[...] * (K - 1.0)
        ib = jnp.clip(jnp.floor(b), 0.0, K - 2.0)
        fb = b - ib
        w00_r[...] = (1.0 - fa) * (1.0 - fb) * val
        w01_r[...] = (1.0 - fa) * fb * val
        w10_r[...] = fa * (1.0 - fb) * val
        w11_r[...] = fa * fb * val
        val_r[...] = val
        base = src_r[...] * KK + ia.astype(jnp.int32) * K + ib.astype(jnp.int32)
        i0_r[...] = base
        i1_r[...] = base + 1
        i2_r[...] = base + K
        i3_r[...] = base + K + 1

    spec = pl.BlockSpec((128, 128), lambda i: (i, 0))
    outs = ([jax.ShapeDtypeStruct((ROWS, 128), jnp.float32)] * 5
            + [jax.ShapeDtypeStruct((ROWS, 128), jnp.int32)] * 4)
    return pl.pallas_call(
        body, grid=(ROWS // 128,),
        in_specs=[spec] * 3, out_specs=[spec] * 9, out_shape=outs,
    )(p0, p1, src)


def _mm(h, wflat):
    n, cin = h.shape
    cols = wflat.shape[1]
    blk = 512

    def body(h_r, w_r, o_r):
        o_r[...] = jnp.dot(h_r[...], w_r[...], preferred_element_type=jnp.float32)

    return pl.pallas_call(
        body, grid=(n // blk,),
        in_specs=[pl.BlockSpec((blk, cin), lambda i: (i, 0)),
                  pl.BlockSpec((cin, cols), lambda i: (0, 0))],
        out_specs=pl.BlockSpec((blk, cols), lambda i: (i, 0)),
        out_shape=jax.ShapeDtypeStruct((n, cols), jnp.float32),
    )(h, wflat)


def _post(aggp, h_prev, root, bias):
    cout = root.shape[1]
    M = aggp.shape[2]
    cin = h_prev.shape[1]
    blk = 1024

    def body(a_r, h_r, r_r, b_r, o_r):
        a = a_r[0] + a_r[1]
        feat = a[:, :cout]
        cnt = a[:, cout:cout + 1]
        v = (feat / jnp.maximum(cnt, 1.0)
             + jnp.dot(h_r[...], r_r[...], preferred_element_type=jnp.float32)
             + b_r[...])
        o_r[...] = jnp.where(v > 0.0, v, jnp.exp(jnp.minimum(v, 0.0)) - 1.0)

    return pl.pallas_call(
        body, grid=(N_PAD // blk,),
        in_specs=[
            pl.BlockSpec((2, blk, M), lambda i: (0, i, 0)),
            pl.BlockSpec((blk, cin), lambda i: (i, 0)),
            pl.BlockSpec((cin, cout), lambda i: (0, 0)),
            pl.BlockSpec((1, cout), lambda i: (0, 0)),
        ],
        out_specs=pl.BlockSpec((blk, cout), lambda i: (i, 0)),
        out_shape=jax.ShapeDtypeStruct((N_PAD, cout), jnp.float32),
    )(aggp, h_prev, root, bias)


def _pool(h3, bat, fc_w, fc_b):
    def body(h_r, b_r, w_r, fb_r, o_r):
        gi = lax.broadcasted_iota(jnp.int32, (NG, N_PAD), 0)
        oh = (gi == b_r[...]).astype(jnp.float32)
        seg = jnp.dot(oh, h_r[...], preferred_element_type=jnp.float32)
        cnt = jnp.sum(oh, axis=1, keepdims=True)
        g = seg / jnp.maximum(cnt, 1.0)
        logits = jnp.dot(g, w_r[...], preferred_element_type=jnp.float32) + fb_r[...]
        m = jnp.max(logits, axis=1, keepdims=True)
        lse = jnp.log(jnp.sum(jnp.exp(logits - m), axis=1, keepdims=True)) + m
        o_r[...] = logits - lse

    return pl.pallas_call(
        body,
        in_specs=[pl.BlockSpec((N_PAD, 64), lambda: (0, 0)),
                  pl.BlockSpec((1, N_PAD), lambda: (0, 0)),
                  pl.BlockSpec((64, 10), lambda: (0, 0)),
                  pl.BlockSpec((1, 10), lambda: (0, 0))],
        out_specs=pl.BlockSpec((NG, 10), lambda: (0, 0)),
        out_shape=jax.ShapeDtypeStruct((NG, 10), jnp.float32),
    )(h3, bat, fc_w, fc_b)


@functools.lru_cache(maxsize=None)
def _make_sc(cout):
    M = cout + 16
    mesh = plsc.VectorSubcoreMesh(core_axis_name="c", subcore_axis_name="s")

    @functools.partial(
        pl.kernel,
        out_type=jax.ShapeDtypeStruct((2, N_PAD, M), jnp.float32),
        mesh=mesh,
        compiler_params=pltpu.CompilerParams(
            needs_layout_passes=False, use_tc_tiling_on_sc=False),
        scratch_types=[
            pltpu.VMEM((EW,), jnp.int32),
            pltpu.VMEM((EW,), jnp.int32),
            pltpu.VMEM((EW,), jnp.int32),
            pltpu.VMEM((EW,), jnp.int32),
            pltpu.VMEM((EW,), jnp.float32),
            pltpu.VMEM((EW,), jnp.float32),
            pltpu.VMEM((EW,), jnp.float32),
            pltpu.VMEM((EW,), jnp.float32),
            pltpu.VMEM((EW,), jnp.float32),
            pltpu.VMEM((NB, B), jnp.int32),
            pltpu.VMEM((B, cout), jnp.float32),
            pltpu.VMEM((B, cout), jnp.float32),
            pltpu.VMEM((B, cout), jnp.float32),
            pltpu.VMEM((B, cout), jnp.float32),
            pltpu.VMEM((B, M), jnp.float32),
            pltpu.VMEM_SHARED((N_PAD, M), jnp.float32),
            pltpu.SemaphoreType.DMA,
        ],
    )
    def sc(Y, i0h, i1h, i2h, i3h, w0h, w1h, w2h, w3h, valh, dsth, zerosh, out,
           i0v, i1v, i2v, i3v, w0v, w1v, w2v, w3v, valv, dstv,
           r0, r1, r2, r3, msg, agg, sem):
        c = lax.axis_index("c")
        s = lax.axis_index("s")
        g = c * 16 + s
        e0 = g * EW
        pltpu.sync_copy(i0h.at[pl.ds(e0, EW)], i0v)
        pltpu.sync_copy(i1h.at[pl.ds(e0, EW)], i1v)
        pltpu.sync_copy(i2h.at[pl.ds(e0, EW)], i2v)
        pltpu.sync_copy(i3h.at[pl.ds(e0, EW)], i3v)
        pltpu.sync_copy(w0h.at[pl.ds(e0, EW)], w0v)
        pltpu.sync_copy(w1h.at[pl.ds(e0, EW)], w1v)
        pltpu.sync_copy(w2h.at[pl.ds(e0, EW)], w2v)
        pltpu.sync_copy(w3h.at[pl.ds(e0, EW)], w3v)
        pltpu.sync_copy(valh.at[pl.ds(e0, EW)], valv)
        pltpu.sync_copy(dsth.at[pl.ds(g * NB, NB)], dstv)
        pltpu.sync_copy(zerosh.at[pl.ds(s * NSL, NSL)], agg.at[pl.ds(s * NSL, NSL)])
        plsc.subcore_barrier()

        zero16 = jnp.zeros((16,), jnp.int32)

        def batch(bi, carry):
            eb = bi * B
            d0 = pltpu.async_copy(Y.at[i0v.at[pl.ds(eb, B)]], r0, sem)
            d1 = pltpu.async_copy(Y.at[i1v.at[pl.ds(eb, B)]], r1, sem)
            d2 = pltpu.async_copy(Y.at[i2v.at[pl.ds(eb, B)]], r2, sem)
            d3 = pltpu.async_copy(Y.at[i3v.at[pl.ds(eb, B)]], r3, sem)
            d0.wait()
            d1.wait()
            d2.wait()
            d3.wait()

            def qloop(q, carry2):
                for j in range(16):
                    b = q * 16 + j
                    ev = zero16 + (eb + b)
                    w0b = plsc.load_gather(w0v, [ev])
                    w1b = plsc.load_gather(w1v, [ev])
                    w2b = plsc.load_gather(w2v, [ev])
                    w3b = plsc.load_gather(w3v, [ev])
                    vb = plsc.load_gather(valv, [ev])
                    for t in range(cout // 16):
                        sl = pl.ds(t * 16, 16)
                        m = (w0b * r0[b, sl] + w1b * r1[b, sl]
                             + w2b * r2[b, sl] + w3b * r3[b, sl])
                        msg[b, sl] = m
                    msg[b, pl.ds(cout, 16)] = vb
                return carry2

            lax.fori_loop(0, B // 16, qloop, 0)
            pltpu.sync_copy(msg, agg.at[dstv.at[bi]], add=True)
            return carry

        lax.fori_loop(0, NB, batch, 0)
        plsc.subcore_barrier()
        pltpu.sync_copy(agg.at[pl.ds(s * NSL, NSL)],
                        out.at[c, pl.ds(s * NSL, NSL)])

    return sc


def kernel(x, position, edge_index, pseudo, batch, W1, root1, b1,
           W2, root2, b2, W3, root3, b3, fc_w, fc_b):
    f32 = jnp.float32
    src = edge_index[0].astype(jnp.int32)
    dst = edge_index[1].astype(jnp.int32)
    pe = E_PAD - E
    p0 = jnp.pad(pseudo[:, 0].astype(f32), (0, pe)).reshape(ROWS, 128)
    p1 = jnp.pad(pseudo[:, 1].astype(f32), (0, pe)).reshape(ROWS, 128)
    src2 = jnp.pad(src, (0, pe)).reshape(ROWS, 128)
    dst2 = jnp.pad(dst, (0, pe)).reshape(NW * NB, B)

    w00, w01, w10, w11, val, i0, i1, i2, i3 = _prep(p0, p1, src2)
    w00, w01, w10, w11, val = (a.reshape(E_PAD) for a in (w00, w01, w10, w11, val))
    i0, i1, i2, i3 = (a.reshape(E_PAD) for a in (i0, i1, i2, i3))

    xp = jnp.pad(x.astype(f32), ((0, N_PAD - N), (0, 0)))
    z48 = jnp.zeros((N_PAD, 48), f32)
    z80 = jnp.zeros((N_PAD, 80), f32)
    w1f = W1.transpose(1, 0, 2).reshape(1, KK * 32)
    w2f = W2.transpose(1, 0, 2).reshape(32, KK * 64)
    w3f = W3.transpose(1, 0, 2).reshape(64, KK * 64)
    sc32 = _make_sc(32)
    sc64 = _make_sc(64)

    Y1 = _mm(xp, w1f).reshape(N_PAD * KK, 32)
    a1 = sc32(Y1, i0, i1, i2, i3, w00, w01, w10, w11, val, dst2, z48)
    h1 = _post(a1, xp, root1, b1.reshape(1, 32))

    Y2 = _mm(h1, w2f).reshape(N_PAD * KK, 64)
    a2 = sc64(Y2, i0, i1, i2, i3, w00, w01, w10, w11, val, dst2, z80)
    h2 = _post(a2, h1, root2, b2.reshape(1, 64))

    Y3 = _mm(h2, w3f).reshape(N_PAD * KK, 64)
    a3 = sc64(Y3, i0, i1, i2, i3, w00, w01, w10, w11, val, dst2, z80)
    h3 = _post(a3, h2, root3, b3.reshape(1, 64))

    bat = jnp.pad(batch.astype(jnp.int32), (0, N_PAD - N),
                  constant_values=NG).reshape(1, N_PAD)
    return _pool(h3, bat, fc_w, fc_b.reshape(1, 10))


# speedup vs baseline: 5.8773x; 1.4390x over previous
import functools

import jax
import jax.numpy as jnp
from jax import lax
from jax.experimental import pallas as pl
from jax.experimental.pallas import tpu as pltpu
from jax.experimental.pallas import tpu_sc as plsc

K = 5
KK = K * K
N = 10000
E = 160000
NG = 64
N_PAD = 10240
E_PAD = 163840
NW = 32
EW = E_PAD // NW
B = 128
NB = EW // B
ROWS = E_PAD // B
VROWS = E // B
NSL = N_PAD // 16


def _prep(p0, p1, src, dst):
    def body(p0_r, p1_r, src_r, dst_r, ei_r, ef_r):
        i = pl.program_id(0)
        rows = i * 128 + lax.broadcasted_iota(jnp.int32, (128, 128), 0)
        val = (rows < VROWS).astype(jnp.float32)
        a = p0_r[...] * (K - 1.0)
        ia = jnp.clip(jnp.floor(a), 0.0, K - 2.0)
        fa = a - ia
        b = p1_r[...] * (K - 1.0)
        ib = jnp.clip(jnp.floor(b), 0.0, K - 2.0)
        fb = b - ib
        ef_r[:, 0, :] = (1.0 - fa) * (1.0 - fb) * val
        ef_r[:, 1, :] = (1.0 - fa) * fb * val
        ef_r[:, 2, :] = fa * (1.0 - fb) * val
        ef_r[:, 3, :] = fa * fb * val
        ef_r[:, 4, :] = val
        base = src_r[...] * KK + ia.astype(jnp.int32) * K + ib.astype(jnp.int32)
        ei_r[:, 0, :] = base
        ei_r[:, 1, :] = base + 1
        ei_r[:, 2, :] = base + K
        ei_r[:, 3, :] = base + K + 1
        ei_r[:, 4, :] = dst_r[...]

    spec = pl.BlockSpec((128, 128), lambda i: (i, 0))
    ospec = pl.BlockSpec((128, 5, 128), lambda i: (i, 0, 0))
    outs = [jax.ShapeDtypeStruct((ROWS, 5, 128), jnp.int32),
            jax.ShapeDtypeStruct((ROWS, 5, 128), jnp.float32)]
    return pl.pallas_call(
        body, grid=(ROWS // 128,),
        in_specs=[spec] * 4, out_specs=[ospec] * 2, out_shape=outs,
    )(p0, p1, src, dst)


def _mm(h, wflat):
    n, cin = h.shape
    cols = wflat.shape[1]
    blk = 512

    def body(h_r, w_r, o_r):
        o_r[...] = jnp.dot(h_r[...], w_r[...], preferred_element_type=jnp.float32)

    return pl.pallas_call(
        body, grid=(n // blk,),
        in_specs=[pl.BlockSpec((blk, cin), lambda i: (i, 0)),
                  pl.BlockSpec((cin, cols), lambda i: (0, 0))],
        out_specs=pl.BlockSpec((blk, cols), lambda i: (i, 0)),
        out_shape=jax.ShapeDtypeStruct((n, cols), jnp.float32),
    )(h, wflat)


def _post(aggp, h_prev, root, bias):
    cout = root.shape[1]
    M = aggp.shape[2]
    cin = h_prev.shape[1]
    blk = 1024

    def body(a_r, h_r, r_r, b_r, o_r):
        a = a_r[0] + a_r[1]
        feat = a[:, :cout]
        cnt = a[:, cout:cout + 1]
        v = (feat / jnp.maximum(cnt, 1.0)
             + jnp.dot(h_r[...], r_r[...], preferred_element_type=jnp.float32)
             + b_r[...])
        o_r[...] = jnp.where(v > 0.0, v, jnp.exp(jnp.minimum(v, 0.0)) - 1.0)

    return pl.pallas_call(
        body, grid=(N_PAD // blk,),
        in_specs=[
            pl.BlockSpec((2, blk, M), lambda i: (0, i, 0)),
            pl.BlockSpec((blk, cin), lambda i: (i, 0)),
            pl.BlockSpec((cin, cout), lambda i: (0, 0)),
            pl.BlockSpec((1, cout), lambda i: (0, 0)),
        ],
        out_specs=pl.BlockSpec((blk, cout), lambda i: (i, 0)),
        out_shape=jax.ShapeDtypeStruct((N_PAD, cout), jnp.float32),
    )(aggp, h_prev, root, bias)


def _pool(h3, bat, fc_w, fc_b):
    def body(h_r, b_r, w_r, fb_r, o_r):
        gi = lax.broadcasted_iota(jnp.int32, (NG, N_PAD), 0)
        oh = (gi == b_r[...]).astype(jnp.float32)
        seg = jnp.dot(oh, h_r[...], preferred_element_type=jnp.float32)
        cnt = jnp.sum(oh, axis=1, keepdims=True)
        g = seg / jnp.maximum(cnt, 1.0)
        logits = jnp.dot(g, w_r[...], preferred_element_type=jnp.float32) + fb_r[...]
        m = jnp.max(logits, axis=1, keepdims=True)
        lse = jnp.log(jnp.sum(jnp.exp(logits - m), axis=1, keepdims=True)) + m
        o_r[...] = logits - lse

    return pl.pallas_call(
        body,
        in_specs=[pl.BlockSpec((N_PAD, 64), lambda: (0, 0)),
                  pl.BlockSpec((1, N_PAD), lambda: (0, 0)),
                  pl.BlockSpec((64, 10), lambda: (0, 0)),
                  pl.BlockSpec((1, 10), lambda: (0, 0))],
        out_specs=pl.BlockSpec((NG, 10), lambda: (0, 0)),
        out_shape=jax.ShapeDtypeStruct((NG, 10), jnp.float32),
    )(h3, bat, fc_w, fc_b)


@functools.lru_cache(maxsize=None)
def _make_sc(cout):
    M = cout + 16
    mesh = plsc.VectorSubcoreMesh(core_axis_name="c", subcore_axis_name="s")

    @functools.partial(
        pl.kernel,
        out_type=jax.ShapeDtypeStruct((2, N_PAD, M), jnp.float32),
        mesh=mesh,
        compiler_params=pltpu.CompilerParams(
            needs_layout_passes=False, use_tc_tiling_on_sc=False),
        scratch_types=[
            pltpu.VMEM((5, B), jnp.int32),
            pltpu.VMEM((5, B), jnp.int32),
            pltpu.VMEM((5, B), jnp.float32),
            pltpu.VMEM((5, B), jnp.float32),
            pltpu.VMEM((B, cout), jnp.float32),
            pltpu.VMEM((B, cout), jnp.float32),
            pltpu.VMEM((B, cout), jnp.float32),
            pltpu.VMEM((B, cout), jnp.float32),
            pltpu.VMEM((B, cout), jnp.float32),
            pltpu.VMEM((B, cout), jnp.float32),
            pltpu.VMEM((B, cout), jnp.float32),
            pltpu.VMEM((B, cout), jnp.float32),
            pltpu.VMEM((B, M), jnp.float32),
            pltpu.VMEM_SHARED((N_PAD, M), jnp.float32),
            pltpu.SemaphoreType.DMA,
            pltpu.SemaphoreType.DMA,
        ],
    )
    def sc(Y, eih, efh, zerosh, out,
           eiv0, eiv1, efv0, efv1,
           r00, r01, r02, r03, r10, r11, r12, r13,
           msg, agg, sem0, sem1):
        c = lax.axis_index("c")
        s = lax.axis_index("s")
        g = c * 16 + s
        gb0 = g * NB
        pltpu.sync_copy(zerosh.at[pl.ds(s * NSL, NSL)], agg.at[pl.ds(s * NSL, NSL)])
        plsc.subcore_barrier()

        bufs = ((eiv0, efv0, (r00, r01, r02, r03), sem0),
                (eiv1, efv1, (r10, r11, r12, r13), sem1))

        def fetch(gb, eiv, efv, rows, sem):
            pltpu.sync_copy(eih.at[gb], eiv)
            pltpu.sync_copy(efh.at[gb], efv)
            for k in range(4):
                pltpu.async_copy(Y.at[eiv.at[k]], rows[k], sem)

        def gwait(eiv, rows, sem):
            for k in range(4):
                pltpu.make_async_copy(Y.at[eiv.at[k]], rows[k], sem).wait()

        fetch(gb0, *bufs[0])

        zero16 = jnp.zeros((16,), jnp.int32)

        def pair(p, carry):
            for buf in range(2):
                bi = p * 2 + buf
                eiv, efv, rows, sem = bufs[buf]
                neiv, nefv, nrows, nsem = bufs[1 - buf]

                @pl.when(bi + 1 < NB)
                def _():
                    fetch(gb0 + bi + 1, neiv, nefv, nrows, nsem)

                gwait(eiv, rows, sem)
                r0, r1, r2, r3 = rows

                def qloop(q, carry2):
                    for j in range(16):
                        b = q * 16 + j
                        ev = zero16 + b
                        w0b = plsc.load_gather(efv.at[0], [ev])
                        w1b = plsc.load_gather(efv.at[1], [ev])
                        w2b = plsc.load_gather(efv.at[2], [ev])
                        w3b = plsc.load_gather(efv.at[3], [ev])
                        vb = plsc.load_gather(efv.at[4], [ev])
                        for t in range(cout // 16):
                            sl = pl.ds(t * 16, 16)
                            m = (w0b * r0[b, sl] + w1b * r1[b, sl]
                                 + w2b * r2[b, sl] + w3b * r3[b, sl])
                            msg[b, sl] = m
                        msg[b, pl.ds(cout, 16)] = vb
                    return carry2

                lax.fori_loop(0, B // 16, qloop, 0)
                pltpu.sync_copy(msg, agg.at[eiv.at[4]], add=True)
            return carry

        lax.fori_loop(0, NB // 2, pair, 0)
        plsc.subcore_barrier()
        pltpu.sync_copy(agg.at[pl.ds(s * NSL, NSL)],
                        out.at[c, pl.ds(s * NSL, NSL)])

    return sc


def kernel(x, position, edge_index, pseudo, batch, W1, root1, b1,
           W2, root2, b2, W3, root3, b3, fc_w, fc_b):
    f32 = jnp.float32
    src = edge_index[0].astype(jnp.int32)
    dst = edge_index[1].astype(jnp.int32)
    pe = E_PAD - E
    p0 = jnp.pad(pseudo[:, 0].astype(f32), (0, pe)).reshape(ROWS, 128)
    p1 = jnp.pad(pseudo[:, 1].astype(f32), (0, pe)).reshape(ROWS, 128)
    src2 = jnp.pad(src, (0, pe)).reshape(ROWS, 128)
    dst2 = jnp.pad(dst, (0, pe)).reshape(ROWS, 128)

    ei, ef = _prep(p0, p1, src2, dst2)

    xp = jnp.pad(x.astype(f32), ((0, N_PAD - N), (0, 0)))
    z48 = jnp.zeros((N_PAD, 48), f32)
    z80 = jnp.zeros((N_PAD, 80), f32)
    w1f = W1.transpose(1, 0, 2).reshape(1, KK * 32)
    w2f = W2.transpose(1, 0, 2).reshape(32, KK * 64)
    w3f = W3.transpose(1, 0, 2).reshape(64, KK * 64)
    sc32 = _make_sc(32)
    sc64 = _make_sc(64)

    Y1 = _mm(xp, w1f).reshape(N_PAD * KK, 32)
    a1 = sc32(Y1, ei, ef, z48)
    h1 = _post(a1, xp, root1, b1.reshape(1, 32))

    Y2 = _mm(h1, w2f).reshape(N_PAD * KK, 64)
    a2 = sc64(Y2, ei, ef, z80)
    h2 = _post(a2, h1, root2, b2.reshape(1, 64))

    Y3 = _mm(h2, w3f).reshape(N_PAD * KK, 64)
    a3 = sc64(Y3, ei, ef, z80)
    h3 = _post(a3, h2, root3, b3.reshape(1, 64))

    bat = jnp.pad(batch.astype(jnp.int32), (0, N_PAD - N),
                  constant_values=NG).reshape(1, N_PAD)
    return _pool(h3, bat, fc_w, fc_b.reshape(1, 10))
